# baseline (speedup 1.0000x reference)
import jax
import jax.numpy as jnp
import numpy as np
from jax import lax
from jax.experimental import pallas as pl

_G0 = float(1.0 / np.sqrt(2.0))
_BLOCK = 6400
_STREAMS = 2


def _mlp(idx, wf, wc, bias, w2row):
    ff = idx.astype(jnp.float32)
    cf = lax.shift_right_arithmetic(idx, 1).astype(jnp.float32)
    pre = (
        jnp.dot(ff, wf, preferred_element_type=jnp.float32)
        + jnp.dot(cf, wc, preferred_element_type=jnp.float32)
        + bias
    )
    h = jnp.tanh(pre)
    return lax.dot_general(
        w2row, h, (((1,), (1,)), ((), ())), preferred_element_type=jnp.float32
    )


def _composer_block(idx0_ref, idx1_ref, wf_ref, wc_ref, bias_ref, w2row_ref,
                    out_ref):
    wf, wc = wf_ref[...], wc_ref[...]
    bias, w2row = bias_ref[...], w2row_ref[...]
    out_ref[0, 0] = _mlp(idx0_ref[...], wf, wc, bias, w2row)[0]
    out_ref[0, 1] = _mlp(idx1_ref[...], wf, wc, bias, w2row)[0]


@jax.jit
def kernel(input_indices, W1, W2):
    n = input_indices.shape[0]
    step_rows = _BLOCK * _STREAMS
    grid = (n + step_rows - 1) // step_rows
    np_rows = grid * step_rows

    z = jnp.zeros((3, 64), jnp.float32)
    wf = jnp.concatenate([W1[:3] * (1.0 / 64.0), z], axis=1)
    wc = jnp.concatenate([z, W1[:3] * (1.0 / 32.0)], axis=1)
    bias = jnp.concatenate([jnp.zeros((1, 64), jnp.float32), W1[3:4]], axis=1)
    w2row = jnp.concatenate([W2, W2 * (_G0 * _G0 * _G0)], axis=0).T

    out = pl.pallas_call(
        _composer_block,
        grid=(grid,),
        in_specs=[
            pl.BlockSpec((_BLOCK, 3), lambda i: (2 * i, 0)),
            pl.BlockSpec((_BLOCK, 3), lambda i: (2 * i + 1, 0)),
            pl.BlockSpec((3, 128), lambda i: (0, 0)),
            pl.BlockSpec((3, 128), lambda i: (0, 0)),
            pl.BlockSpec((1, 128), lambda i: (0, 0)),
            pl.BlockSpec((1, 128), lambda i: (0, 0)),
        ],
        out_specs=pl.BlockSpec((1, _STREAMS, _BLOCK), lambda i: (i, 0, 0)),
        out_shape=jax.ShapeDtypeStruct((grid, _STREAMS, _BLOCK), jnp.float32),
    )(input_indices, input_indices, wf, wc, bias, w2row)
    return out.reshape(np_rows, 1)[:n]


# speedup vs baseline: 2.4299x; 2.4299x over previous
import jax
import jax.numpy as jnp
import numpy as np
from jax import lax
from jax.experimental import pallas as pl

_G0 = float(1.0 / np.sqrt(2.0))
_BLOCK = 12800


def _composer_block(idx_ref, wft_ref, wct_ref, bias_ref, w2row_ref, out_ref):
    idx = idx_ref[...]
    ff = idx.astype(jnp.float32)
    cf = lax.shift_right_arithmetic(idx, 1).astype(jnp.float32)
    pre = (
        jnp.dot(wft_ref[...], ff, preferred_element_type=jnp.float32)
        + jnp.dot(wct_ref[...], cf, preferred_element_type=jnp.float32)
        + bias_ref[...]
    )
    h = jnp.tanh(pre)
    out_ref[0] = jnp.dot(
        w2row_ref[...], h, preferred_element_type=jnp.float32
    )


@jax.jit
def kernel(input_indices, W1, W2):
    n = input_indices.shape[0]
    grid = (n + _BLOCK - 1) // _BLOCK
    np_rows = grid * _BLOCK
    idx_t = input_indices.T

    z = jnp.zeros((3, 64), jnp.float32)
    wft = jnp.concatenate([W1[:3] * (1.0 / 64.0), z], axis=1).T
    wct = jnp.concatenate([z, W1[:3] * (1.0 / 32.0)], axis=1).T
    bias = jnp.concatenate(
        [jnp.zeros((1, 64), jnp.float32), W1[3:4]], axis=1).T
    w2row = jnp.concatenate([W2, W2 * (_G0 * _G0 * _G0)], axis=0).T

    out = pl.pallas_call(
        _composer_block,
        grid=(grid,),
        in_specs=[
            pl.BlockSpec((3, _BLOCK), lambda i: (0, i)),
            pl.BlockSpec((128, 3), lambda i: (0, 0)),
            pl.BlockSpec((128, 3), lambda i: (0, 0)),
            pl.BlockSpec((128, 1), lambda i: (0, 0)),
            pl.BlockSpec((1, 128), lambda i: (0, 0)),
        ],
        out_specs=pl.BlockSpec((1, 1, _BLOCK), lambda i: (i, 0, 0)),
        out_shape=jax.ShapeDtypeStruct((grid, 1, _BLOCK), jnp.float32),
    )(idx_t, wft, wct, bias, w2row)
    return out.reshape(np_rows, 1)[:n]


# speedup vs baseline: 3.5359x; 1.4552x over previous
import jax
import jax.numpy as jnp
import numpy as np
from jax import lax
from jax.experimental import pallas as pl

_G0 = float(1.0 / np.sqrt(2.0))
_BLOCK = 25600


def _composer_block(idx_ref, w_ref, w2row_ref, out_ref):
    idx = idx_ref[...]
    ff = idx.astype(jnp.float32)
    cf = lax.shift_right_arithmetic(idx, 1).astype(jnp.float32)
    ones = jnp.ones((1, idx.shape[1]), jnp.float32)
    feats = jnp.concatenate([ff, cf, ones], axis=0)
    pre = jnp.dot(w_ref[...], feats, preferred_element_type=jnp.float32)
    h = jnp.tanh(pre)
    out_ref[0] = jnp.dot(
        w2row_ref[...], h, preferred_element_type=jnp.float32)


@jax.jit
def kernel(input_indices, W1, W2):
    n = input_indices.shape[0]
    grid = (n + _BLOCK - 1) // _BLOCK
    np_rows = grid * _BLOCK
    idx_t = input_indices.T

    z = jnp.zeros((3, 64), jnp.float32)
    wft = jnp.concatenate([W1[:3] * (1.0 / 64.0), z], axis=1).T
    wct = jnp.concatenate([z, W1[:3] * (1.0 / 32.0)], axis=1).T
    bias = jnp.concatenate(
        [jnp.zeros((1, 64), jnp.float32), W1[3:4]], axis=1).T
    w_all = jnp.concatenate([wft, wct, bias], axis=1)
    w2row = jnp.concatenate([W2, W2 * (_G0 * _G0 * _G0)], axis=0).T

    out = pl.pallas_call(
        _composer_block,
        grid=(grid,),
        in_specs=[
            pl.BlockSpec((3, _BLOCK), lambda i: (0, i)),
            pl.BlockSpec((128, 7), lambda i: (0, 0)),
            pl.BlockSpec((1, 128), lambda i: (0, 0)),
        ],
        out_specs=pl.BlockSpec((1, 1, _BLOCK), lambda i: (i, 0, 0)),
        out_shape=jax.ShapeDtypeStruct((grid, 1, _BLOCK), jnp.float32),
    )(idx_t, w_all, w2row)
    return out.reshape(np_rows, 1)[:n]
